# baseline (speedup 1.0000x reference)
import functools

import jax
import jax.numpy as jnp
from jax import lax
from jax.experimental import pallas as pl
from jax.experimental.pallas import tpu as pltpu
from jax.experimental.pallas import tpu_sc as plsc

NUM_ENTITIES = 1000000
ENTITY_DIM = 32
BATCH = 16384
ROWS_PER_VIEW = 4
VIEW_DIM = ROWS_PER_VIEW * ENTITY_DIM
NUM_VIEW_ROWS = NUM_ENTITIES // ROWS_PER_VIEW

_INFO = plsc.get_sparse_core_info()
NC = _INFO.num_cores
NS = _INFO.num_subcores
NW = NC * NS
B_PER_W = BATCH // NW
IDX_CHUNK = 128
N_CHUNKS = B_PER_W // IDX_CHUNK
LANES = 16


@functools.partial(
    pl.kernel,
    mesh=plsc.VectorSubcoreMesh(core_axis_name="c", subcore_axis_name="s"),
    out_type=jax.ShapeDtypeStruct((BATCH, ENTITY_DIM), jnp.float32),
    scratch_types=[
        pltpu.VMEM((B_PER_W,), jnp.int32),
        pltpu.VMEM((B_PER_W,), jnp.int32),
        pltpu.VMEM((2, IDX_CHUNK, VIEW_DIM), jnp.float32),
        pltpu.VMEM((B_PER_W, ENTITY_DIM), jnp.float32),
        pltpu.SemaphoreType.DMA,
        pltpu.SemaphoreType.DMA,
    ],
)
def _gather_sc(idx_hbm, table_hbm, out_hbm, idx_v, grp_v, rows_v, out_v,
               sem0, sem1):
    wid = lax.axis_index("s") * NC + lax.axis_index("c")
    base = wid * B_PER_W
    sems = (sem0, sem1)

    pltpu.sync_copy(idx_hbm.at[pl.ds(base, B_PER_W)], idx_v)

    @pl.loop(0, B_PER_W // LANES)
    def _compute_groups(g):
        v = idx_v[pl.ds(g * LANES, LANES)]
        grp_v[pl.ds(g * LANES, LANES)] = lax.shift_right_logical(v, 2)

    def start_chunk(c, buf):
        return pltpu.async_copy(
            table_hbm.at[grp_v.at[pl.ds(c * IDX_CHUNK, IDX_CHUNK)]],
            rows_v.at[buf],
            sems[buf],
        )

    copy = start_chunk(0, 0)
    for c in range(N_CHUNKS):
        buf = c % 2
        next_copy = start_chunk(c + 1, 1 - buf) if c + 1 < N_CHUNKS else None
        copy.wait()

        @pl.loop(0, IDX_CHUNK // LANES)
        def _extract(g):
            svec = (idx_v[pl.ds(c * IDX_CHUNK + g * LANES, LANES)] & 3) * (
                ENTITY_DIM
            )
            for l in range(LANES):
                i = g * LANES + l
                s = svec[l]
                for h in range(ENTITY_DIM // LANES):
                    out_v[c * IDX_CHUNK + i, pl.ds(h * LANES, LANES)] = (
                        rows_v[buf, i, pl.ds(s + h * LANES, LANES)]
                    )

        copy = next_copy

    pltpu.sync_copy(out_v, out_hbm.at[pl.ds(base, B_PER_W)])


def kernel(indexes, entity_table):
    table_view = entity_table.reshape(NUM_VIEW_ROWS, VIEW_DIM)
    return _gather_sc(indexes.astype(jnp.int32), table_view)


# speedup vs baseline: 1.0032x; 1.0032x over previous
import functools

import jax
import jax.numpy as jnp
from jax import lax
from jax.experimental import pallas as pl
from jax.experimental.pallas import tpu as pltpu
from jax.experimental.pallas import tpu_sc as plsc

NUM_ENTITIES = 1000000
ENTITY_DIM = 32
BATCH = 16384

_INFO = plsc.get_sparse_core_info()
NC = _INFO.num_cores
NS = _INFO.num_subcores
NW = NC * NS
B_PER_W = BATCH // NW
IDX_CHUNK = 128
N_CHUNKS = B_PER_W // IDX_CHUNK


@functools.partial(
    pl.kernel,
    mesh=plsc.VectorSubcoreMesh(core_axis_name="c", subcore_axis_name="s"),
    compiler_params=pltpu.CompilerParams(use_tc_tiling_on_sc=False),
    out_type=jax.ShapeDtypeStruct((BATCH, ENTITY_DIM), jnp.float32),
    scratch_types=[
        pltpu.VMEM((B_PER_W,), jnp.int32),
        pltpu.VMEM((B_PER_W, ENTITY_DIM), jnp.float32),
        pltpu.SemaphoreType.DMA,
    ],
)
def _gather_sc(idx_hbm, table_hbm, out_hbm, idx_v, rows_v, sem):
    wid = lax.axis_index("s") * NC + lax.axis_index("c")
    base = wid * B_PER_W

    pltpu.sync_copy(idx_hbm.at[pl.ds(base, B_PER_W)], idx_v)

    copies = [
        pltpu.async_copy(
            table_hbm.at[idx_v.at[pl.ds(c * IDX_CHUNK, IDX_CHUNK)]],
            rows_v.at[pl.ds(c * IDX_CHUNK, IDX_CHUNK), :],
            sem,
        )
        for c in range(N_CHUNKS)
    ]
    for copy in copies:
        copy.wait()

    pltpu.sync_copy(rows_v, out_hbm.at[pl.ds(base, B_PER_W)])


def kernel(indexes, entity_table):
    return _gather_sc(indexes.astype(jnp.int32), entity_table)
